# baseline (speedup 1.0000x reference)
import functools

import jax
import jax.numpy as jnp
from jax import lax
from jax.experimental import pallas as pl
from jax.experimental.pallas import tpu as pltpu
from jax.experimental.pallas import tpu_sc as plsc

N = 10000
NP = 10240
D = 128
E = 320000
NC = 2
NS = 16
NW = NC * NS
EP = E // NW
C = 80
G = EP // C
SB = 5
GB = G // SB
SL = NP // NS
MB = 512


def _rsqrt_nr(d):
  bits = lax.bitcast_convert_type(d, jnp.int32)
  y = lax.bitcast_convert_type(
      jnp.int32(0x5F3759DF) - (bits >> 1), jnp.float32)
  h = 0.5 * d
  for _ in range(3):
    y = y * (1.5 - h * y * y)
  return y


def _deg_kernel(dst_hbm, ew_hbm, degp_hbm, dstv, eww, zb, deg_sh):
  c = lax.axis_index("c")
  s = lax.axis_index("s")
  wid = c * NS + s

  def zset(i, _):
    zb[pl.ds(i * 16, 16)] = jnp.zeros((16,), jnp.float32)
    return 0
  lax.fori_loop(0, SL // 16, zset, 0)
  pltpu.sync_copy(zb, deg_sh.at[pl.ds(s * SL, SL)])
  plsc.subcore_barrier()

  def sblock(b, _):
    pltpu.sync_copy(dst_hbm.at[wid, b], dstv)
    pltpu.sync_copy(ew_hbm.at[wid, b], eww)

    def body(g, _):
      pltpu.sync_copy(eww.at[g], deg_sh.at[dstv.at[g]], add=True)
      return 0
    lax.fori_loop(0, GB, body, 0)
    return 0
  lax.fori_loop(0, SB, sblock, 0)
  plsc.subcore_barrier()

  sl = pl.ds(s * SL, SL)
  pltpu.sync_copy(deg_sh.at[sl], degp_hbm.at[c].at[sl])


def _edge_kernel(f_hbm, src_hbm, dst_hbm, ew_hbm, degp_hbm, aggp_hbm,
                 src2, dst2, ew2, dinv, rows, rowsb,
                 gsem0, gsem1, ssem0, ssem1, agg_sh):
  c = lax.axis_index("c")
  s = lax.axis_index("s")
  wid = c * NS + s

  def dstage(k, _):
    pltpu.sync_copy(degp_hbm.at[0, k], rows.at[pl.ds(0, 5)])
    pltpu.sync_copy(degp_hbm.at[1, k], rows.at[pl.ds(8, 5)])

    def dloop(i, _):
      sl = pl.ds((i % 8) * 16, 16)
      d = rows[i // 8, sl] + rows[8 + i // 8, sl] + 1.0
      dinv[pl.ds(k * 640 + i * 16, 16)] = _rsqrt_nr(d)
      return 0
    lax.fori_loop(0, 40, dloop, 0)
    return 0
  lax.fori_loop(0, NS, dstage, 0)

  def zset(i, _):
    rows[i // (D // 16), pl.ds((i % (D // 16)) * 16, 16)] = (
        jnp.zeros((16,), jnp.float32))
    return 0
  lax.fori_loop(0, C * (D // 16), zset, 0)

  def zcopy(k, _):
    pltpu.sync_copy(rows, agg_sh.at[pl.ds(s * SL + k * C, C)])
    return 0
  lax.fori_loop(0, SL // C, zcopy, 0)
  plsc.subcore_barrier()

  bufs = (rows, rowsb)
  gsems = (gsem0, gsem1)
  ssems = (ssem0, ssem1)

  def sblock(b, _):
    pltpu.sync_copy(src_hbm.at[wid, b], src2)
    pltpu.sync_copy(dst_hbm.at[wid, b], dst2)
    pltpu.sync_copy(ew_hbm.at[wid, b], ew2)

    def nloop(g, _):
      for k in range(C // 16):
        sl = pl.ds(k * 16, 16)
        a = plsc.load_gather(dinv, [src2[g, sl]])
        bb = plsc.load_gather(dinv, [dst2[g, sl]])
        ew2[g, sl] = a * ew2[g, sl] * bb
      return 0
    lax.fori_loop(0, GB, nloop, 0)

    H = C // 2

    def _gather(g, i):
      pltpu.async_copy(
          f_hbm.at[src2.at[g, pl.ds(0, H)]], bufs[i].at[pl.ds(0, H)],
          gsems[i])
      pltpu.async_copy(
          f_hbm.at[src2.at[g, pl.ds(H, H)]], bufs[i].at[pl.ds(H, H)],
          gsems[i])

    def _gwait(g, i):
      pltpu.make_async_copy(
          f_hbm.at[src2.at[g, pl.ds(0, H)]], bufs[i].at[pl.ds(0, H)],
          gsems[i]).wait()
      pltpu.make_async_copy(
          f_hbm.at[src2.at[g, pl.ds(H, H)]], bufs[i].at[pl.ds(H, H)],
          gsems[i]).wait()

    _gather(0, 0)
    for g in range(GB):
      i = g % 2
      ni = (g + 1) % 2
      rb = bufs[i]
      _gwait(g, i)

      @plsc.parallel_loop(0, C)
      def scale(r):
        nrm = plsc.load_gather(
            ew2, [jnp.full((16,), g, jnp.int32),
                  jnp.full((16,), r, jnp.int32)])
        for j in range(D // 16):
          sl = pl.ds(j * 16, 16)
          rb[r, sl] = rb[r, sl] * nrm

      pltpu.async_copy(rb, agg_sh.at[dst2.at[g]], ssems[i], add=True)
      if g + 1 < GB:
        if g - 1 >= 0:
          pltpu.make_async_copy(
              bufs[ni], agg_sh.at[dst2.at[g - 1]], ssems[ni]).wait()
        _gather(g + 1, ni)
    for g in range(GB - 2, GB):
      i = g % 2
      pltpu.make_async_copy(
          bufs[i], agg_sh.at[dst2.at[g]], ssems[i]).wait()
    return 0
  lax.fori_loop(0, SB, sblock, 0)
  plsc.subcore_barrier()

  def wout(k, _):
    sl = pl.ds(s * SL + k * C, C)
    pltpu.sync_copy(agg_sh.at[sl], aggp_hbm.at[c].at[sl])
    return 0
  lax.fori_loop(0, SL // C, wout, 0)


_sc_kernels = None


def _get_sc_kernels():
  global _sc_kernels
  if _sc_kernels is None:
    mesh = plsc.VectorSubcoreMesh(
        core_axis_name="c", subcore_axis_name="s",
        num_cores=NC, num_subcores=NS)
    params = pltpu.CompilerParams(needs_layout_passes=False)
    deg = pl.kernel(
        _deg_kernel,
        out_type=jax.ShapeDtypeStruct((NC, NP), jnp.float32),
        mesh=mesh,
        compiler_params=params,
        scratch_types=[
            pltpu.VMEM((GB, C), jnp.int32),
            pltpu.VMEM((GB, C), jnp.float32),
            pltpu.VMEM((SL,), jnp.float32),
            pltpu.VMEM_SHARED((NP,), jnp.float32),
        ],
    )
    edge = pl.kernel(
        _edge_kernel,
        out_type=jax.ShapeDtypeStruct((NC, NP, D), jnp.float32),
        mesh=mesh,
        compiler_params=params,
        scratch_types=[
            pltpu.VMEM((GB, C), jnp.int32),
            pltpu.VMEM((GB, C), jnp.int32),
            pltpu.VMEM((GB, C), jnp.float32),
            pltpu.VMEM((NP,), jnp.float32),
            pltpu.VMEM((C, D), jnp.float32),
            pltpu.VMEM((C, D), jnp.float32),
            pltpu.SemaphoreType.DMA,
            pltpu.SemaphoreType.DMA,
            pltpu.SemaphoreType.DMA,
            pltpu.SemaphoreType.DMA,
            pltpu.VMEM_SHARED((NP, D), jnp.float32),
        ],
    )
    _sc_kernels = (deg, edge)
  return _sc_kernels


def _mm_kernel(relu, aggp_ref, f_ref, degp_ref, w_ref, b_ref, o_ref):
  d = degp_ref[0] + degp_ref[1]
  coef = 1.0 / (d + 1.0)
  acc = aggp_ref[0] + aggp_ref[1] + f_ref[...] * coef
  y = jnp.dot(acc, w_ref[...], preferred_element_type=jnp.float32) + b_ref[...]
  if relu:
    y = jnp.maximum(y, 0.0)
  o_ref[...] = y


def _layer_mm(aggp, f, degp3, w, b, relu):
  return pl.pallas_call(
      functools.partial(_mm_kernel, relu),
      grid=(NP // MB,),
      in_specs=[
          pl.BlockSpec((2, MB, D), lambda i: (0, i, 0)),
          pl.BlockSpec((MB, D), lambda i: (i, 0)),
          pl.BlockSpec((2, MB, 1), lambda i: (0, i, 0)),
          pl.BlockSpec((D, D), lambda i: (0, 0)),
          pl.BlockSpec((1, D), lambda i: (0, 0)),
      ],
      out_specs=pl.BlockSpec((MB, D), lambda i: (i, 0)),
      out_shape=jax.ShapeDtypeStruct((NP, D), jnp.float32),
  )(aggp, f, degp3, w, b)


def kernel(x, edge_weight, W1, b1, W2, b2, edge_index):
  src = edge_index[0].reshape(NW, SB, GB, C)
  dst = edge_index[1].reshape(NW, SB, GB, C)
  ew = edge_weight.reshape(NW, SB, GB, C)
  xp = jnp.zeros((NP, D), jnp.float32).at[:N].set(x)

  _deg, _edge = _get_sc_kernels()
  degp = _deg(dst, ew)
  degp3 = degp.reshape(NC, NP, 1)
  degp4 = degp.reshape(NC, NS, 5, 128)

  aggp1 = _edge(xp, src, dst, ew, degp4)
  h = _layer_mm(aggp1, xp, degp3, W1, b1.reshape(1, D), relu=True)

  aggp2 = _edge(h, src, dst, ew, degp4)
  out = _layer_mm(aggp2, h, degp3, W2, b2.reshape(1, D), relu=False)
  return out[:N]


# speedup vs baseline: 1.2328x; 1.2328x over previous
import functools

import jax
import jax.numpy as jnp
import numpy as np
from jax import lax
from jax.experimental import pallas as pl
from jax.experimental.pallas import tpu as pltpu
from jax.experimental.pallas import tpu_sc as plsc

N = 10000
NP = 10240
D = 128
E = 320000
NC = 2
NS = 16
NW = NC * NS
EP = E // NW
C = 80
G = EP // C
SB = 5
GB = G // SB
SL = NP // NS
MB = 512


def _rsqrt_nr(d):
  bits = lax.bitcast_convert_type(d, jnp.int32)
  y = lax.bitcast_convert_type(
      jnp.int32(0x5F3759DF) - (bits >> 1), jnp.float32)
  h = 0.5 * d
  for _ in range(3):
    y = y * (1.5 - h * y * y)
  return y


def _deg_kernel(dst_hbm, ew_hbm, degp_hbm, dstv, eww, zb, deg_sh):
  c = lax.axis_index("c")
  s = lax.axis_index("s")
  wid = c * NS + s

  def zset(i, _):
    zb[pl.ds(i * 16, 16)] = jnp.zeros((16,), jnp.float32)
    return 0
  lax.fori_loop(0, SL // 16, zset, 0)
  pltpu.sync_copy(zb, deg_sh.at[pl.ds(s * SL, SL)])
  plsc.subcore_barrier()

  def sblock(b, _):
    pltpu.sync_copy(dst_hbm.at[wid, b], dstv)
    pltpu.sync_copy(ew_hbm.at[wid, b], eww)

    def body(g, _):
      pltpu.sync_copy(eww.at[g], deg_sh.at[dstv.at[g]], add=True)
      return 0
    lax.fori_loop(0, GB, body, 0)
    return 0
  lax.fori_loop(0, SB, sblock, 0)
  plsc.subcore_barrier()

  sl = pl.ds(s * SL, SL)
  pltpu.sync_copy(deg_sh.at[sl], degp_hbm.at[c].at[sl])


def _edge_kernel(f_hbm, src_hbm, dst_hbm, ew_hbm, degp_hbm, aggp_hbm,
                 src2, dst2, ew2, dinv, rows, rowsb, gb0, gb1,
                 gsem0, gsem1, ssem0, ssem1, agg_sh):
  c = lax.axis_index("c")
  s = lax.axis_index("s")
  wid = c * NS + s

  def dstage(k, _):
    pltpu.sync_copy(degp_hbm.at[0, k], rows.at[pl.ds(0, 5)])
    pltpu.sync_copy(degp_hbm.at[1, k], rows.at[pl.ds(8, 5)])

    def dloop(i, _):
      sl = pl.ds((i % 8) * 16, 16)
      d = rows[i // 8, sl] + rows[8 + i // 8, sl] + 1.0
      dinv[pl.ds(k * 640 + i * 16, 16)] = _rsqrt_nr(d)
      return 0
    lax.fori_loop(0, 40, dloop, 0)
    return 0
  lax.fori_loop(0, NS, dstage, 0)

  def zset(i, _):
    rows[i // (D // 16), pl.ds((i % (D // 16)) * 16, 16)] = (
        jnp.zeros((16,), jnp.float32))
    return 0
  lax.fori_loop(0, C * (D // 16), zset, 0)

  def zcopy(k, _):
    pltpu.sync_copy(rows, agg_sh.at[pl.ds(s * SL + k * C, C)])
    return 0
  lax.fori_loop(0, SL // C, zcopy, 0)
  plsc.subcore_barrier()

  bufs = (rows, rowsb)
  gbufs = (gb0, gb1)
  gsems = (gsem0, gsem1)
  ssems = (ssem0, ssem1)

  def sblock(b, _):
    pltpu.sync_copy(src_hbm.at[wid, b], src2)
    pltpu.sync_copy(dst_hbm.at[wid, b], dst2)
    pltpu.sync_copy(ew_hbm.at[wid, b], ew2)

    def nloop(g, _):
      for k in range(C // 16):
        sl = pl.ds(k * 16, 16)
        a = plsc.load_gather(dinv, [src2[g, sl]])
        bb = plsc.load_gather(dinv, [dst2[g, sl]])
        ew2[g, sl] = a * ew2[g, sl] * bb
      return 0
    lax.fori_loop(0, GB, nloop, 0)

    pltpu.async_copy(f_hbm.at[src2.at[0]], gbufs[0], gsems[0])
    for g in range(GB):
      i = g % 2
      ni = (g + 1) % 2
      gb = gbufs[i]
      sb = bufs[i]
      pltpu.make_async_copy(f_hbm.at[src2.at[g]], gb, gsems[i]).wait()
      if g + 1 < GB:
        pltpu.async_copy(f_hbm.at[src2.at[g + 1]], gbufs[ni], gsems[ni])
      if g - 2 >= 0:
        pltpu.make_async_copy(
            sb, agg_sh.at[dst2.at[g]], ssems[i]).wait()

      @plsc.parallel_loop(0, C)
      def scale(r):
        nrm = plsc.load_gather(
            ew2, [jnp.full((16,), g, jnp.int32),
                  jnp.full((16,), r, jnp.int32)])
        for j in range(D // 32):
          w = gb[r, pl.ds(j * 16, 16)]
          lo = lax.bitcast_convert_type(w << 16, jnp.float32)
          hi = lax.bitcast_convert_type(w & jnp.int32(-65536), jnp.float32)
          sb[r, pl.ds(j * 32, 16)] = lo * nrm
          sb[r, pl.ds(j * 32 + 16, 16)] = hi * nrm

      pltpu.async_copy(sb, agg_sh.at[dst2.at[g]], ssems[i], add=True)
    for g in range(GB - 2, GB):
      i = g % 2
      pltpu.make_async_copy(
          bufs[i], agg_sh.at[dst2.at[g]], ssems[i]).wait()
    return 0
  lax.fori_loop(0, SB, sblock, 0)
  plsc.subcore_barrier()

  def wout(k, _):
    sl = pl.ds(s * SL + k * C, C)
    pltpu.sync_copy(agg_sh.at[sl], aggp_hbm.at[c].at[sl])
    return 0
  lax.fori_loop(0, SL // C, wout, 0)


_sc_kernels = None


def _get_sc_kernels():
  global _sc_kernels
  if _sc_kernels is None:
    mesh = plsc.VectorSubcoreMesh(
        core_axis_name="c", subcore_axis_name="s",
        num_cores=NC, num_subcores=NS)
    params = pltpu.CompilerParams(needs_layout_passes=False, use_tc_tiling_on_sc=False)
    deg = pl.kernel(
        _deg_kernel,
        out_type=jax.ShapeDtypeStruct((NC, NP), jnp.float32),
        mesh=mesh,
        compiler_params=params,
        scratch_types=[
            pltpu.VMEM((GB, C), jnp.int32),
            pltpu.VMEM((GB, C), jnp.float32),
            pltpu.VMEM((SL,), jnp.float32),
            pltpu.VMEM_SHARED((NP,), jnp.float32),
        ],
    )
    edge = pl.kernel(
        _edge_kernel,
        out_type=jax.ShapeDtypeStruct((NC, NP, D), jnp.float32),
        mesh=mesh,
        compiler_params=params,
        scratch_types=[
            pltpu.VMEM((GB, C), jnp.int32),
            pltpu.VMEM((GB, C), jnp.int32),
            pltpu.VMEM((GB, C), jnp.float32),
            pltpu.VMEM((NP,), jnp.float32),
            pltpu.VMEM((C, D), jnp.float32),
            pltpu.VMEM((C, D), jnp.float32),
            pltpu.VMEM((C, D // 2), jnp.int32),
            pltpu.VMEM((C, D // 2), jnp.int32),
            pltpu.SemaphoreType.DMA,
            pltpu.SemaphoreType.DMA,
            pltpu.SemaphoreType.DMA,
            pltpu.SemaphoreType.DMA,
            pltpu.VMEM_SHARED((NP, D), jnp.float32),
        ],
    )
    _sc_kernels = (deg, edge)
  return _sc_kernels


def _mm_kernel(relu, aggp_ref, f_ref, degp_ref, w_ref, b_ref, o_ref):
  d = degp_ref[0] + degp_ref[1]
  coef = 1.0 / (d + 1.0)
  acc = aggp_ref[0] + aggp_ref[1] + f_ref[...] * coef
  y = jnp.dot(acc, w_ref[...], preferred_element_type=jnp.float32) + b_ref[...]
  if relu:
    y = jnp.maximum(y, 0.0)
  o_ref[...] = y


def _layer_mm(aggp, f, degp3, w, b, relu):
  return pl.pallas_call(
      functools.partial(_mm_kernel, relu),
      grid=(NP // MB,),
      in_specs=[
          pl.BlockSpec((2, MB, D), lambda i: (0, i, 0)),
          pl.BlockSpec((MB, D), lambda i: (i, 0)),
          pl.BlockSpec((2, MB, 1), lambda i: (0, i, 0)),
          pl.BlockSpec((D, D), lambda i: (0, 0)),
          pl.BlockSpec((1, D), lambda i: (0, 0)),
      ],
      out_specs=pl.BlockSpec((MB, D), lambda i: (i, 0)),
      out_shape=jax.ShapeDtypeStruct((NP, D), jnp.float32),
  )(aggp, f, degp3, w, b)


_PPERM = np.empty((D,), np.int32)
for _j in range(D // 32):
  for _k in range(16):
    _PPERM[32 * _j + 2 * _k] = 32 * _j + _k
    _PPERM[32 * _j + 2 * _k + 1] = 32 * _j + 16 + _k


def _pack_bf16(f):
  fb = f.astype(jnp.bfloat16)[:, _PPERM]
  return lax.bitcast_convert_type(fb.reshape(NP, D // 2, 2), jnp.int32)


def kernel(x, edge_weight, W1, b1, W2, b2, edge_index):
  src = edge_index[0].reshape(NW, SB, GB, C)
  dst = edge_index[1].reshape(NW, SB, GB, C)
  ew = edge_weight.reshape(NW, SB, GB, C)
  xp = jnp.zeros((NP, D), jnp.float32).at[:N].set(x)

  _deg, _edge = _get_sc_kernels()
  degp = _deg(dst, ew)
  degp3 = degp.reshape(NC, NP, 1)
  degp4 = degp.reshape(NC, NS, 5, 128)

  aggp1 = _edge(_pack_bf16(xp), src, dst, ew, degp4)
  h = _layer_mm(aggp1, xp, degp3, W1, b1.reshape(1, D), relu=True)

  aggp2 = _edge(_pack_bf16(h), src, dst, ew, degp4)
  out = _layer_mm(aggp2, h, degp3, W2, b2.reshape(1, D), relu=False)
  return out[:N]


# speedup vs baseline: 1.2640x; 1.0253x over previous
import functools

import jax
import jax.numpy as jnp
import numpy as np
from jax import lax
from jax.experimental import pallas as pl
from jax.experimental.pallas import tpu as pltpu
from jax.experimental.pallas import tpu_sc as plsc

N = 10000
NP = 10240
D = 128
E = 320000
NC = 2
NS = 16
NW = NC * NS
EP = E // NW
C = 80
G = EP // C
SB = 5
GB = G // SB
SL = NP // NS
MB = 512


def _rsqrt_nr(d):
  bits = lax.bitcast_convert_type(d, jnp.int32)
  y = lax.bitcast_convert_type(
      jnp.int32(0x5F3759DF) - (bits >> 1), jnp.float32)
  h = 0.5 * d
  for _ in range(3):
    y = y * (1.5 - h * y * y)
  return y


def _deg_kernel(dst_hbm, ew_hbm, degp_hbm, dstv, eww, zb, deg_sh):
  c = lax.axis_index("c")
  s = lax.axis_index("s")
  wid = c * NS + s

  def zset(i, _):
    zb[pl.ds(i * 16, 16)] = jnp.zeros((16,), jnp.float32)
    return 0
  lax.fori_loop(0, SL // 16, zset, 0)
  pltpu.sync_copy(zb, deg_sh.at[pl.ds(s * SL, SL)])
  plsc.subcore_barrier()

  def sblock(b, _):
    pltpu.sync_copy(dst_hbm.at[wid, b], dstv)
    pltpu.sync_copy(ew_hbm.at[wid, b], eww)

    def body(g, _):
      pltpu.sync_copy(eww.at[g], deg_sh.at[dstv.at[g]], add=True)
      return 0
    lax.fori_loop(0, GB, body, 0)
    return 0
  lax.fori_loop(0, SB, sblock, 0)
  plsc.subcore_barrier()

  sl = pl.ds(s * SL, SL)
  pltpu.sync_copy(deg_sh.at[sl], degp_hbm.at[c].at[sl])


def _edge_kernel(f_hbm, src_hbm, dst_hbm, ew_hbm, degp_hbm, aggp_hbm,
                 src2, dst2, ew2, dinv, rows, rowsb, gb0, gb1,
                 gsem0, gsem1, ssem0, ssem1, agg_sh):
  c = lax.axis_index("c")
  s = lax.axis_index("s")
  wid = c * NS + s

  def dstage(k, _):
    pltpu.sync_copy(degp_hbm.at[0, k], rows.at[pl.ds(0, 5)])
    pltpu.sync_copy(degp_hbm.at[1, k], rows.at[pl.ds(8, 5)])

    def dloop(i, _):
      sl = pl.ds((i % 8) * 16, 16)
      d = rows[i // 8, sl] + rows[8 + i // 8, sl] + 1.0
      dinv[pl.ds(k * 640 + i * 16, 16)] = _rsqrt_nr(d)
      return 0
    lax.fori_loop(0, 40, dloop, 0)
    return 0
  lax.fori_loop(0, NS, dstage, 0)

  def zset(i, _):
    rows[i // (D // 16), pl.ds((i % (D // 16)) * 16, 16)] = (
        jnp.zeros((16,), jnp.float32))
    return 0
  lax.fori_loop(0, C * (D // 16), zset, 0)

  def zcopy(k, _):
    pltpu.sync_copy(rows, agg_sh.at[pl.ds(s * SL + k * C, C)])
    return 0
  lax.fori_loop(0, SL // C, zcopy, 0)
  plsc.subcore_barrier()

  bufs = (rows, rowsb)
  gbufs = (gb0, gb1)
  gsems = (gsem0, gsem1)
  ssems = (ssem0, ssem1)

  def sblock(b, _):
    pltpu.sync_copy(src_hbm.at[wid, b], src2)
    pltpu.sync_copy(dst_hbm.at[wid, b], dst2)
    pltpu.sync_copy(ew_hbm.at[wid, b], ew2)

    def nloop(g, _):
      for k in range(C // 16):
        sl = pl.ds(k * 16, 16)
        a = plsc.load_gather(dinv, [src2[g, sl]])
        bb = plsc.load_gather(dinv, [dst2[g, sl]])
        ew2[g, sl] = a * ew2[g, sl] * bb
      return 0
    lax.fori_loop(0, GB, nloop, 0)

    pltpu.async_copy(f_hbm.at[src2.at[0]], gbufs[0], gsems[0])
    for g in range(GB):
      i = g % 2
      ni = (g + 1) % 2
      gb = gbufs[i]
      sb = bufs[i]
      pltpu.make_async_copy(f_hbm.at[src2.at[g]], gb, gsems[i]).wait()
      if g + 1 < GB:
        pltpu.async_copy(f_hbm.at[src2.at[g + 1]], gbufs[ni], gsems[ni])
      if g - 2 >= 0:
        pltpu.make_async_copy(
            sb, agg_sh.at[dst2.at[g]], ssems[i]).wait()

      @plsc.parallel_loop(0, C)
      def scale(r):
        nrm = plsc.load_gather(
            ew2, [jnp.full((16,), g, jnp.int32),
                  jnp.full((16,), r, jnp.int32)])
        for j in range(D // 32):
          w = gb[r, pl.ds(j * 16, 16)]
          lo = lax.bitcast_convert_type(w << 16, jnp.float32)
          hi = lax.bitcast_convert_type(w & jnp.int32(-65536), jnp.float32)
          sb[r, pl.ds(j * 32, 16)] = lo * nrm
          sb[r, pl.ds(j * 32 + 16, 16)] = hi * nrm

      pltpu.async_copy(sb, agg_sh.at[dst2.at[g]], ssems[i], add=True)
    for g in range(GB - 2, GB):
      i = g % 2
      pltpu.make_async_copy(
          bufs[i], agg_sh.at[dst2.at[g]], ssems[i]).wait()
    return 0
  lax.fori_loop(0, SB, sblock, 0)
  plsc.subcore_barrier()

  def wout(k, _):
    sl = pl.ds(s * SL + k * C, C)
    pltpu.sync_copy(agg_sh.at[sl], aggp_hbm.at[c].at[sl])
    return 0
  lax.fori_loop(0, SL // C, wout, 0)


_sc_kernels = None


def _get_sc_kernels():
  global _sc_kernels
  if _sc_kernels is None:
    mesh = plsc.VectorSubcoreMesh(
        core_axis_name="c", subcore_axis_name="s",
        num_cores=NC, num_subcores=NS)
    params = pltpu.CompilerParams(needs_layout_passes=False, use_tc_tiling_on_sc=False)
    deg = pl.kernel(
        _deg_kernel,
        out_type=jax.ShapeDtypeStruct((NC, NP), jnp.float32),
        mesh=mesh,
        compiler_params=params,
        scratch_types=[
            pltpu.VMEM((GB, C), jnp.int32),
            pltpu.VMEM((GB, C), jnp.float32),
            pltpu.VMEM((SL,), jnp.float32),
            pltpu.VMEM_SHARED((NP,), jnp.float32),
        ],
    )
    edge = pl.kernel(
        _edge_kernel,
        out_type=jax.ShapeDtypeStruct((NC, NP, D), jnp.float32),
        mesh=mesh,
        compiler_params=params,
        scratch_types=[
            pltpu.VMEM((GB, C), jnp.int32),
            pltpu.VMEM((GB, C), jnp.int32),
            pltpu.VMEM((GB, C), jnp.float32),
            pltpu.VMEM((NP,), jnp.float32),
            pltpu.VMEM((C, D), jnp.float32),
            pltpu.VMEM((C, D), jnp.float32),
            pltpu.VMEM((C, D // 2), jnp.int32),
            pltpu.VMEM((C, D // 2), jnp.int32),
            pltpu.SemaphoreType.DMA,
            pltpu.SemaphoreType.DMA,
            pltpu.SemaphoreType.DMA,
            pltpu.SemaphoreType.DMA,
            pltpu.VMEM_SHARED((NP, D), jnp.float32),
        ],
    )
    _sc_kernels = (deg, edge)
  return _sc_kernels


def _mm_kernel(relu, aggp_ref, f_ref, degp_ref, w_ref, b_ref, o_ref):
  d = degp_ref[0] + degp_ref[1]
  coef = 1.0 / (d + 1.0)
  acc = aggp_ref[0] + aggp_ref[1] + f_ref[...] * coef
  y = jnp.dot(acc, w_ref[...], preferred_element_type=jnp.float32) + b_ref[...]
  if relu:
    y = jnp.maximum(y, 0.0)
  o_ref[...] = y


def _layer_mm(aggp, f, degp3, w, b, relu):
  return pl.pallas_call(
      functools.partial(_mm_kernel, relu),
      grid=(NP // MB,),
      in_specs=[
          pl.BlockSpec((2, MB, D), lambda i: (0, i, 0)),
          pl.BlockSpec((MB, D), lambda i: (i, 0)),
          pl.BlockSpec((2, MB, 1), lambda i: (0, i, 0)),
          pl.BlockSpec((D, D), lambda i: (0, 0)),
          pl.BlockSpec((1, D), lambda i: (0, 0)),
      ],
      out_specs=pl.BlockSpec((MB, D), lambda i: (i, 0)),
      out_shape=jax.ShapeDtypeStruct((NP, D), jnp.float32),
  )(aggp, f, degp3, w, b)


_PPERM = np.empty((D,), np.int32)
for _j in range(D // 32):
  for _k in range(16):
    _PPERM[32 * _j + 2 * _k] = 32 * _j + _k
    _PPERM[32 * _j + 2 * _k + 1] = 32 * _j + 16 + _k


def _pack_bf16(f):
  fb = f.astype(jnp.bfloat16)
  return lax.bitcast_convert_type(fb.reshape(NP, D // 2, 2), jnp.int32)


def kernel(x, edge_weight, W1, b1, W2, b2, edge_index):
  src = edge_index[0].reshape(NW, SB, GB, C)
  dst = edge_index[1].reshape(NW, SB, GB, C)
  ew = edge_weight.reshape(NW, SB, GB, C)
  xp = jnp.zeros((NP, D), jnp.float32).at[:N].set(x)

  _deg, _edge = _get_sc_kernels()
  degp = _deg(dst, ew)
  degp3 = degp.reshape(NC, NP, 1)
  degp4 = degp.reshape(NC, NS, 5, 128)

  aggp1 = _edge(_pack_bf16(xp), src, dst, ew, degp4)
  h = _layer_mm(aggp1, xp, degp3, W1, b1.reshape(1, D), relu=True)

  aggp2 = _edge(_pack_bf16(h), src, dst, ew, degp4)
  out = _layer_mm(aggp2, h, degp3, W2, b2.reshape(1, D), relu=False)
  return out[:N]


# speedup vs baseline: 1.3169x; 1.0418x over previous
import functools

import jax
import jax.numpy as jnp
import numpy as np
from jax import lax
from jax.experimental import pallas as pl
from jax.experimental.pallas import tpu as pltpu
from jax.experimental.pallas import tpu_sc as plsc

N = 10000
NP = 10240
D = 128
E = 320000
NC = 2
NS = 16
NW = NC * NS
EP = E // NW
C = 80
G = EP // C
SB = 5
GB = G // SB
SL = NP // NS
MB = 512


def _rsqrt_nr(d):
  bits = lax.bitcast_convert_type(d, jnp.int32)
  y = lax.bitcast_convert_type(
      jnp.int32(0x5F3759DF) - (bits >> 1), jnp.float32)
  h = 0.5 * d
  for _ in range(3):
    y = y * (1.5 - h * y * y)
  return y


def _deg_kernel(dst_hbm, ew_hbm, degp_hbm, dstv, eww, zb, deg_sh):
  c = lax.axis_index("c")
  s = lax.axis_index("s")
  wid = c * NS + s

  def zset(i, _):
    zb[pl.ds(i * 16, 16)] = jnp.zeros((16,), jnp.float32)
    return 0
  lax.fori_loop(0, SL // 16, zset, 0)
  pltpu.sync_copy(zb, deg_sh.at[pl.ds(s * SL, SL)])
  plsc.subcore_barrier()

  def sblock(b, _):
    pltpu.sync_copy(dst_hbm.at[wid, b], dstv)
    pltpu.sync_copy(ew_hbm.at[wid, b], eww)

    def body(g, _):
      pltpu.sync_copy(eww.at[g], deg_sh.at[dstv.at[g]], add=True)
      return 0
    lax.fori_loop(0, GB, body, 0)
    return 0
  lax.fori_loop(0, SB, sblock, 0)
  plsc.subcore_barrier()

  sl = pl.ds(s * SL, SL)
  pltpu.sync_copy(deg_sh.at[sl], degp_hbm.at[c].at[sl])


def _edge_kernel(f_hbm, src_hbm, dst_hbm, ew_hbm, aggp_hbm,
                 src2, dst2, ew2, rows, rowsb, rowsc, gb0, gb1,
                 gsem0, gsem1, ssem0, ssem1, ssem2, agg_sh):
  c = lax.axis_index("c")
  s = lax.axis_index("s")
  wid = c * NS + s

  def zset(i, _):
    rows[i // (D // 16), pl.ds((i % (D // 16)) * 16, 16)] = (
        jnp.zeros((16,), jnp.float32))
    return 0
  lax.fori_loop(0, C * (D // 16), zset, 0)

  def zcopy(k, _):
    pltpu.sync_copy(rows, agg_sh.at[pl.ds(s * SL + k * C, C)])
    return 0
  lax.fori_loop(0, SL // C, zcopy, 0)
  plsc.subcore_barrier()

  bufs = (rows, rowsb, rowsc)
  gbufs = (gb0, gb1)
  gsems = (gsem0, gsem1)
  ssems = (ssem0, ssem1, ssem2)

  def sblock(b, _):
    pltpu.sync_copy(src_hbm.at[wid, b], src2)
    pltpu.sync_copy(dst_hbm.at[wid, b], dst2)
    pltpu.sync_copy(ew_hbm.at[wid, b], ew2)

    pltpu.async_copy(f_hbm.at[src2.at[0]], gbufs[0], gsems[0])
    for g in range(GB):
      gi = g % 2
      gni = (g + 1) % 2
      si = g % 3
      gb = gbufs[gi]
      sb = bufs[si]
      pltpu.make_async_copy(f_hbm.at[src2.at[g]], gb, gsems[gi]).wait()
      if g + 1 < GB:
        pltpu.async_copy(f_hbm.at[src2.at[g + 1]], gbufs[gni], gsems[gni])
      if g - 3 >= 0:
        pltpu.make_async_copy(
            sb, agg_sh.at[dst2.at[g]], ssems[si]).wait()

      @plsc.parallel_loop(0, C)
      def scale(r):
        nrm = plsc.load_gather(
            ew2, [jnp.full((16,), g, jnp.int32),
                  jnp.full((16,), r, jnp.int32)])
        for j in range(D // 32):
          w = gb[r, pl.ds(j * 16, 16)]
          lo = lax.bitcast_convert_type(w << 16, jnp.float32)
          hi = lax.bitcast_convert_type(w & jnp.int32(-65536), jnp.float32)
          sb[r, pl.ds(j * 32, 16)] = lo * nrm
          sb[r, pl.ds(j * 32 + 16, 16)] = hi * nrm

      pltpu.async_copy(sb, agg_sh.at[dst2.at[g]], ssems[si], add=True)
    for g in range(GB - 3, GB):
      si = g % 3
      pltpu.make_async_copy(
          bufs[si], agg_sh.at[dst2.at[g]], ssems[si]).wait()
    return 0
  lax.fori_loop(0, SB, sblock, 0)
  plsc.subcore_barrier()

  def wout(k, _):
    sl = pl.ds(s * SL + k * C, C)
    pltpu.sync_copy(agg_sh.at[sl], aggp_hbm.at[c].at[sl])
    return 0
  lax.fori_loop(0, SL // C, wout, 0)


_sc_kernels = None


def _get_sc_kernels():
  global _sc_kernels
  if _sc_kernels is None:
    mesh = plsc.VectorSubcoreMesh(
        core_axis_name="c", subcore_axis_name="s",
        num_cores=NC, num_subcores=NS)
    params = pltpu.CompilerParams(needs_layout_passes=False, use_tc_tiling_on_sc=False)
    deg = pl.kernel(
        _deg_kernel,
        out_type=jax.ShapeDtypeStruct((NC, NP), jnp.float32),
        mesh=mesh,
        compiler_params=params,
        scratch_types=[
            pltpu.VMEM((GB, C), jnp.int32),
            pltpu.VMEM((GB, C), jnp.float32),
            pltpu.VMEM((SL,), jnp.float32),
            pltpu.VMEM_SHARED((NP,), jnp.float32),
        ],
    )
    edge = pl.kernel(
        _edge_kernel,
        out_type=jax.ShapeDtypeStruct((NC, NP, D), jnp.float32),
        mesh=mesh,
        compiler_params=params,
        scratch_types=[
            pltpu.VMEM((GB, C), jnp.int32),
            pltpu.VMEM((GB, C), jnp.int32),
            pltpu.VMEM((GB, C), jnp.float32),
            pltpu.VMEM((C, D), jnp.float32),
            pltpu.VMEM((C, D), jnp.float32),
            pltpu.VMEM((C, D), jnp.float32),
            pltpu.VMEM((C, D // 2), jnp.int32),
            pltpu.VMEM((C, D // 2), jnp.int32),
            pltpu.SemaphoreType.DMA,
            pltpu.SemaphoreType.DMA,
            pltpu.SemaphoreType.DMA,
            pltpu.SemaphoreType.DMA,
            pltpu.SemaphoreType.DMA,
            pltpu.VMEM_SHARED((NP, D), jnp.float32),
        ],
    )
    _sc_kernels = (deg, edge)
  return _sc_kernels


def _mm_kernel(relu, emit_scaled, aggp_ref, y_ref, degp_ref, w_ref, b_ref,
               o_ref):
  dinv = lax.rsqrt(degp_ref[0] + degp_ref[1] + 1.0)
  acc = (aggp_ref[0] + aggp_ref[1] + y_ref[...]) * dinv
  z = jnp.dot(acc, w_ref[...], preferred_element_type=jnp.float32) + b_ref[...]
  if relu:
    z = jnp.maximum(z, 0.0)
  if emit_scaled:
    z = z * dinv
  o_ref[...] = z


def _dscale_kernel(x_ref, degp_ref, o_ref):
  dinv = lax.rsqrt(degp_ref[0] + degp_ref[1] + 1.0)
  o_ref[...] = x_ref[...] * dinv


def _dscale(x, degp3):
  return pl.pallas_call(
      _dscale_kernel,
      grid=(NP // MB,),
      in_specs=[
          pl.BlockSpec((MB, D), lambda i: (i, 0)),
          pl.BlockSpec((2, MB, 1), lambda i: (0, i, 0)),
      ],
      out_specs=pl.BlockSpec((MB, D), lambda i: (i, 0)),
      out_shape=jax.ShapeDtypeStruct((NP, D), jnp.float32),
  )(x, degp3)


def _layer_mm(aggp, f, degp3, w, b, relu, emit_scaled):
  return pl.pallas_call(
      functools.partial(_mm_kernel, relu, emit_scaled),
      grid=(NP // MB,),
      in_specs=[
          pl.BlockSpec((2, MB, D), lambda i: (0, i, 0)),
          pl.BlockSpec((MB, D), lambda i: (i, 0)),
          pl.BlockSpec((2, MB, 1), lambda i: (0, i, 0)),
          pl.BlockSpec((D, D), lambda i: (0, 0)),
          pl.BlockSpec((1, D), lambda i: (0, 0)),
      ],
      out_specs=pl.BlockSpec((MB, D), lambda i: (i, 0)),
      out_shape=jax.ShapeDtypeStruct((NP, D), jnp.float32),
  )(aggp, f, degp3, w, b)


_PPERM = np.empty((D,), np.int32)
for _j in range(D // 32):
  for _k in range(16):
    _PPERM[32 * _j + 2 * _k] = 32 * _j + _k
    _PPERM[32 * _j + 2 * _k + 1] = 32 * _j + 16 + _k


def _pack_bf16(f):
  fb = f.astype(jnp.bfloat16)[:, _PPERM]
  return lax.bitcast_convert_type(fb.reshape(NP, D // 2, 2), jnp.int32)


def kernel(x, edge_weight, W1, b1, W2, b2, edge_index):
  src = edge_index[0].reshape(NW, SB, GB, C)
  dst = edge_index[1].reshape(NW, SB, GB, C)
  ew = edge_weight.reshape(NW, SB, GB, C)
  xp = jnp.zeros((NP, D), jnp.float32).at[:N].set(x)

  _deg, _edge = _get_sc_kernels()
  degp = _deg(dst, ew)
  degp3 = degp.reshape(NC, NP, 1)

  y1 = _dscale(xp, degp3)
  aggp1 = _edge(_pack_bf16(y1), src, dst, ew)
  y2 = _layer_mm(aggp1, y1, degp3, W1, b1.reshape(1, D),
                 relu=True, emit_scaled=True)

  aggp2 = _edge(_pack_bf16(y2), src, dst, ew)
  out = _layer_mm(aggp2, y2, degp3, W2, b2.reshape(1, D),
                  relu=False, emit_scaled=False)
  return out[:N]


# speedup vs baseline: 1.4546x; 1.1046x over previous
import functools

import jax
import jax.numpy as jnp
import numpy as np
from jax import lax
from jax.experimental import pallas as pl
from jax.experimental.pallas import tpu as pltpu
from jax.experimental.pallas import tpu_sc as plsc

N = 10000
NP = 10240
D = 128
E = 320000
NC = 2
NS = 16
NW = NC * NS
EP = E // NW
C = 80
G = EP // C
SB = 5
GB = G // SB
SL = NP // NS
MB = 512


def _rsqrt_nr(d):
  bits = lax.bitcast_convert_type(d, jnp.int32)
  y = lax.bitcast_convert_type(
      jnp.int32(0x5F3759DF) - (bits >> 1), jnp.float32)
  h = 0.5 * d
  for _ in range(3):
    y = y * (1.5 - h * y * y)
  return y


def _deg_kernel(dst_hbm, ew_hbm, degp_hbm, dstv, eww, zb, deg_sh):
  c = lax.axis_index("c")
  s = lax.axis_index("s")
  wid = c * NS + s

  def zset(i, _):
    zb[pl.ds(i * 16, 16)] = jnp.zeros((16,), jnp.float32)
    return 0
  lax.fori_loop(0, SL // 16, zset, 0)
  pltpu.sync_copy(zb, deg_sh.at[pl.ds(s * SL, SL)])
  plsc.subcore_barrier()

  def sblock(b, _):
    pltpu.sync_copy(dst_hbm.at[wid, b], dstv)
    pltpu.sync_copy(ew_hbm.at[wid, b], eww)

    def body(g, _):
      pltpu.sync_copy(eww.at[g], deg_sh.at[dstv.at[g]], add=True)
      return 0
    lax.fori_loop(0, GB, body, 0)
    return 0
  lax.fori_loop(0, SB, sblock, 0)
  plsc.subcore_barrier()

  sl = pl.ds(s * SL, SL)
  pltpu.sync_copy(deg_sh.at[sl], degp_hbm.at[c].at[sl])


def _edge_kernel(f_hbm, src_hbm, dst_hbm, ew_hbm, aggp_hbm,
                 src2, dst2, ew2, rows, rowsb, gb0, gb1, gb2,
                 gsem0, gsem1, gsem2, ssem0, ssem1, agg_sh):
  c = lax.axis_index("c")
  s = lax.axis_index("s")
  wid = c * NS + s

  def zset(i, _):
    rows[i // (D // 16), pl.ds((i % (D // 16)) * 16, 16)] = (
        jnp.zeros((16,), jnp.float32))
    return 0
  lax.fori_loop(0, C * (D // 16), zset, 0)

  def zcopy(k, _):
    pltpu.sync_copy(rows, agg_sh.at[pl.ds(s * SL + k * C, C)])
    return 0
  lax.fori_loop(0, SL // C, zcopy, 0)
  plsc.subcore_barrier()

  bufs = (rows, rowsb)
  gbufs = (gb0, gb1, gb2)
  gsems = (gsem0, gsem1, gsem2)
  ssems = (ssem0, ssem1)

  def sblock(b, _):
    pltpu.sync_copy(src_hbm.at[wid, b], src2)
    pltpu.sync_copy(dst_hbm.at[wid, b], dst2)
    pltpu.sync_copy(ew_hbm.at[wid, b], ew2)

    pltpu.async_copy(f_hbm.at[src2.at[0]], gbufs[0], gsems[0])
    pltpu.async_copy(f_hbm.at[src2.at[1]], gbufs[1], gsems[1])
    for g in range(GB):
      gi = g % 3
      si = g % 2
      gb = gbufs[gi]
      sb = bufs[si]
      pltpu.make_async_copy(f_hbm.at[src2.at[g]], gb, gsems[gi]).wait()
      if g + 2 < GB:
        gni = (g + 2) % 3
        pltpu.async_copy(f_hbm.at[src2.at[g + 2]], gbufs[gni], gsems[gni])
      if g - 2 >= 0:
        pltpu.make_async_copy(
            sb, agg_sh.at[dst2.at[g]], ssems[si]).wait()

      @plsc.parallel_loop(0, C)
      def scale(r):
        nrm = plsc.load_gather(
            ew2, [jnp.full((16,), g, jnp.int32),
                  jnp.full((16,), r, jnp.int32)])
        for j in range(D // 32):
          w = gb[r, pl.ds(j * 16, 16)]
          lo = lax.bitcast_convert_type(w << 16, jnp.float32)
          hi = lax.bitcast_convert_type(w & jnp.int32(-65536), jnp.float32)
          sb[r, pl.ds(j * 32, 16)] = lo * nrm
          sb[r, pl.ds(j * 32 + 16, 16)] = hi * nrm

      pltpu.async_copy(sb, agg_sh.at[dst2.at[g]], ssems[si], add=True)
    for g in range(GB - 2, GB):
      si = g % 2
      pltpu.make_async_copy(
          bufs[si], agg_sh.at[dst2.at[g]], ssems[si]).wait()
    return 0
  lax.fori_loop(0, SB, sblock, 0)
  plsc.subcore_barrier()

  def wout(k, _):
    sl = pl.ds(s * SL + k * C, C)
    pltpu.sync_copy(agg_sh.at[sl], aggp_hbm.at[c].at[sl])
    return 0
  lax.fori_loop(0, SL // C, wout, 0)


_sc_kernels = None


def _get_sc_kernels():
  global _sc_kernels
  if _sc_kernels is None:
    mesh = plsc.VectorSubcoreMesh(
        core_axis_name="c", subcore_axis_name="s",
        num_cores=NC, num_subcores=NS)
    params = pltpu.CompilerParams(needs_layout_passes=False, use_tc_tiling_on_sc=False)
    deg = pl.kernel(
        _deg_kernel,
        out_type=jax.ShapeDtypeStruct((NC, NP), jnp.float32),
        mesh=mesh,
        compiler_params=params,
        scratch_types=[
            pltpu.VMEM((GB, C), jnp.int32),
            pltpu.VMEM((GB, C), jnp.float32),
            pltpu.VMEM((SL,), jnp.float32),
            pltpu.VMEM_SHARED((NP,), jnp.float32),
        ],
    )
    edge = pl.kernel(
        _edge_kernel,
        out_type=jax.ShapeDtypeStruct((NC, NP, D), jnp.float32),
        mesh=mesh,
        compiler_params=params,
        scratch_types=[
            pltpu.VMEM((GB, C), jnp.int32),
            pltpu.VMEM((GB, C), jnp.int32),
            pltpu.VMEM((GB, C), jnp.float32),
            pltpu.VMEM((C, D), jnp.float32),
            pltpu.VMEM((C, D), jnp.float32),
            pltpu.VMEM((C, D // 2), jnp.int32),
            pltpu.VMEM((C, D // 2), jnp.int32),
            pltpu.VMEM((C, D // 2), jnp.int32),
            pltpu.SemaphoreType.DMA,
            pltpu.SemaphoreType.DMA,
            pltpu.SemaphoreType.DMA,
            pltpu.SemaphoreType.DMA,
            pltpu.SemaphoreType.DMA,
            pltpu.VMEM_SHARED((NP, D), jnp.float32),
        ],
    )
    _sc_kernels = (deg, edge)
  return _sc_kernels


def _mm_kernel(relu, emit_scaled, aggp_ref, y_ref, degp_ref, w_ref, b_ref,
               o_ref):
  dinv = lax.rsqrt(degp_ref[0] + degp_ref[1] + 1.0)
  acc = (aggp_ref[0] + aggp_ref[1] + y_ref[...]) * dinv
  z = jnp.dot(acc, w_ref[...], preferred_element_type=jnp.float32) + b_ref[...]
  if relu:
    z = jnp.maximum(z, 0.0)
  if emit_scaled:
    z = z * dinv
  o_ref[...] = z


def _dscale_kernel(x_ref, degp_ref, o_ref):
  dinv = lax.rsqrt(degp_ref[0] + degp_ref[1] + 1.0)
  o_ref[...] = x_ref[...] * dinv


def _dscale(x, degp3):
  return pl.pallas_call(
      _dscale_kernel,
      grid=(NP // MB,),
      in_specs=[
          pl.BlockSpec((MB, D), lambda i: (i, 0)),
          pl.BlockSpec((2, MB, 1), lambda i: (0, i, 0)),
      ],
      out_specs=pl.BlockSpec((MB, D), lambda i: (i, 0)),
      out_shape=jax.ShapeDtypeStruct((NP, D), jnp.float32),
  )(x, degp3)


def _layer_mm(aggp, f, degp3, w, b, relu, emit_scaled):
  return pl.pallas_call(
      functools.partial(_mm_kernel, relu, emit_scaled),
      grid=(NP // MB,),
      in_specs=[
          pl.BlockSpec((2, MB, D), lambda i: (0, i, 0)),
          pl.BlockSpec((MB, D), lambda i: (i, 0)),
          pl.BlockSpec((2, MB, 1), lambda i: (0, i, 0)),
          pl.BlockSpec((D, D), lambda i: (0, 0)),
          pl.BlockSpec((1, D), lambda i: (0, 0)),
      ],
      out_specs=pl.BlockSpec((MB, D), lambda i: (i, 0)),
      out_shape=jax.ShapeDtypeStruct((NP, D), jnp.float32),
  )(aggp, f, degp3, w, b)


_PPERM = np.empty((D,), np.int32)
for _j in range(D // 32):
  for _k in range(16):
    _PPERM[32 * _j + 2 * _k] = 32 * _j + _k
    _PPERM[32 * _j + 2 * _k + 1] = 32 * _j + 16 + _k


def _pack_bf16(f):
  fb = f.astype(jnp.bfloat16)[:, _PPERM]
  return lax.bitcast_convert_type(fb.reshape(NP, D // 2, 2), jnp.int32)


def kernel(x, edge_weight, W1, b1, W2, b2, edge_index):
  src = edge_index[0].reshape(NW, SB, GB, C)
  dst = edge_index[1].reshape(NW, SB, GB, C)
  ew = edge_weight.reshape(NW, SB, GB, C)
  xp = jnp.zeros((NP, D), jnp.float32).at[:N].set(x)

  _deg, _edge = _get_sc_kernels()
  degp = _deg(dst, ew)
  degp3 = degp.reshape(NC, NP, 1)

  y1 = _dscale(xp, degp3)
  aggp1 = _edge(_pack_bf16(y1), src, dst, ew)
  y2 = _layer_mm(aggp1, y1, degp3, W1, b1.reshape(1, D),
                 relu=True, emit_scaled=True)

  aggp2 = _edge(_pack_bf16(y2), src, dst, ew)
  out = _layer_mm(aggp2, y2, degp3, W2, b2.reshape(1, D),
                  relu=False, emit_scaled=False)
  return out[:N]
